# baseline (speedup 1.0000x reference)
import functools

import jax
import jax.numpy as jnp
from jax import lax
from jax.experimental import pallas as pl
from jax.experimental.pallas import tpu as pltpu
from jax.experimental.pallas import tpu_sc as plsc

_B = 4096
_L = 20
_D = 64
_F = 4

_NC = 2
_NS = 16
_NW = _NC * _NS

_S = _F * _B
_SPW = _S // _NW
_CHUNK = 32
_RPC = _CHUNK * _L
_NSTREAM = _RPC // 128
_NCHUNK = _SPW // _CHUNK


def _sc_body(table_hbm, idx_hbm, out_hbm, idx_v, rows_v, out_v, sems):
    wid = lax.axis_index("s") * _NC + lax.axis_index("c")
    idx_rows_per_worker = _NCHUNK * _NSTREAM

    pltpu.sync_copy(idx_hbm.at[pl.ds(wid * idx_rows_per_worker, idx_rows_per_worker)], idx_v)

    def load_and_fire(c, buf):
        for k in range(_NSTREAM):
            pltpu.async_copy(
                table_hbm.at[idx_v.at[c * _NSTREAM + k]],
                rows_v.at[buf, pl.ds(k * 128, 128)],
                sems.at[buf],
            )

    def drain(c, buf):
        for k in range(_NSTREAM):
            pltpu.make_async_copy(
                table_hbm.at[idx_v.at[c * _NSTREAM + k]],
                rows_v.at[buf, pl.ds(k * 128, 128)],
                sems.at[buf],
            ).wait()

    def pool_and_store(c, buf):
        def s_body(s, _):
            r0 = s * _L
            for j in range(_D // 16):
                dsl = pl.ds(j * 16, 16)
                acc = rows_v[buf, r0, dsl]
                for l in range(1, _L):
                    acc = acc + rows_v[buf, r0 + l, dsl]
                out_v[s, dsl] = acc
            return 0

        lax.fori_loop(0, _CHUNK, s_body, 0)
        pltpu.sync_copy(out_v, out_hbm.at[pl.ds(wid * _SPW + c * _CHUNK, _CHUNK)])

    load_and_fire(0, 0)

    def chunk_body(c, _):
        buf = lax.rem(c, 2)
        nxt = 1 - buf

        @pl.when(c + 1 < _NCHUNK)
        def _():
            load_and_fire(c + 1, nxt)

        drain(c, buf)
        pool_and_store(c, buf)
        return 0

    lax.fori_loop(0, _NCHUNK, chunk_body, 0)


def _make_sc_call():
    mesh = plsc.VectorSubcoreMesh(core_axis_name="c", subcore_axis_name="s")
    return pl.kernel(
        _sc_body,
        mesh=mesh,
        compiler_params=pltpu.CompilerParams(use_tc_tiling_on_sc=False),
        out_type=jax.ShapeDtypeStruct((_S, _D), jnp.float32),
        scratch_types=[
            pltpu.VMEM((_NCHUNK * _NSTREAM, 128), jnp.int32),
            pltpu.VMEM((2, _RPC, _D), jnp.float32),
            pltpu.VMEM((_CHUNK, _D), jnp.float32),
            pltpu.SemaphoreType.DMA((2,)),
        ],
    )


@jax.jit
def kernel(head_idx, tail_idx, neg_head_idx, neg_tail_idx, entity_table):
    idx_all = jnp.stack([head_idx, tail_idx, neg_head_idx, neg_tail_idx])
    idx_flat = idx_all.reshape(_F * _B * _L // 128, 128)
    out = _make_sc_call()(entity_table, idx_flat)
    return out.reshape(_F, _B, _D)
